# baseline (speedup 1.0000x reference)
import functools

import jax
import jax.numpy as jnp
from jax import lax
from jax.experimental import pallas as pl
from jax.experimental.pallas import tpu as pltpu
from jax.experimental.pallas import tpu_sc as plsc

_EMBED_DIM = 64
_NUM_CORES = 2
_NUM_SUBCORES = 16
_NUM_WORKERS = _NUM_CORES * _NUM_SUBCORES
_LANE = 128
_SUB = 8


@functools.lru_cache(maxsize=None)
def _build_gather(hist: int, batch: int, n_class: int):
    n_j = batch // _LANE
    jw = n_j // _NUM_WORKERS
    n_a = _EMBED_DIM // _SUB
    n_blk = hist * jw
    assert n_blk % 2 == 0
    mesh = plsc.VectorSubcoreMesh(core_axis_name="c", subcore_axis_name="s")

    @functools.partial(
        pl.kernel,
        out_type=jax.ShapeDtypeStruct(
            (hist, n_a, n_j, _SUB, _LANE), jnp.float32
        ),
        mesh=mesh,
        compiler_params=pltpu.CompilerParams(
            use_tc_tiling_on_sc=False, needs_layout_passes=False
        ),
        scratch_types=[
            pltpu.VMEM((hist, jw * _LANE), jnp.int32),
            pltpu.VMEM((_LANE, _EMBED_DIM), jnp.float32),
            pltpu.VMEM((_LANE, _EMBED_DIM), jnp.float32),
            pltpu.VMEM((n_a, _SUB, _LANE), jnp.float32),
            pltpu.VMEM((n_a, _SUB, _LANE), jnp.float32),
            pltpu.SemaphoreType.DMA,
            pltpu.SemaphoreType.DMA,
            pltpu.SemaphoreType.DMA,
            pltpu.SemaphoreType.DMA,
        ],
    )
    def gather_kernel(
        xt_hbm, table_hbm, out_hbm,
        idx_v, g0, g1, t0, t1, sem_g0, sem_g1, sem_o0, sem_o1,
    ):
        wid = lax.axis_index("s") * _NUM_CORES + lax.axis_index("c")
        g_refs = (g0, g1)
        t_refs = (t0, t1)
        sem_g = (sem_g0, sem_g1)
        sem_o = (sem_o0, sem_o1)

        pltpu.sync_copy(
            xt_hbm.at[:, pl.ds(wid * (jw * _LANE), jw * _LANE)], idx_v
        )

        def idx_slice(b):
            h = b // jw
            mcol = lax.rem(b, jw)
            return idx_v.at[h, pl.ds(mcol * _LANE, _LANE)]

        for m in range(2):
            pltpu.async_copy(
                table_hbm.at[idx_slice(m)], g_refs[m], sem_g[m]
            )

        rows = [
            lax.iota(jnp.int32, 16) + jnp.int32(16 * kk) for kk in range(8)
        ]
        cols = [
            jnp.full((16,), col, jnp.int32) for col in range(_EMBED_DIM)
        ]

        def body(g, carry):
            for m in range(2):
                b = 2 * g + m
                h = b // jw
                mcol = lax.rem(b, jw)
                bcol = wid * jw + mcol

                @pl.when(b >= 2)
                def _():
                    for a in range(n_a):
                        pltpu.make_async_copy(
                            t_refs[m].at[a], out_hbm.at[h, a, bcol], sem_o[m]
                        ).wait()

                pltpu.make_async_copy(
                    table_hbm.at[idx_slice(b)], g_refs[m], sem_g[m]
                ).wait()

                for a in range(n_a):
                    for c in range(_SUB):
                        col = cols[a * _SUB + c]
                        for kk in range(8):
                            v = plsc.load_gather(g_refs[m], [rows[kk], col])
                            t_refs[m][a, c, pl.ds(16 * kk, 16)] = v

                for a in range(n_a):
                    pltpu.async_copy(
                        t_refs[m].at[a], out_hbm.at[h, a, bcol], sem_o[m]
                    )

                @pl.when(b + 2 < n_blk)
                def _():
                    pltpu.async_copy(
                        table_hbm.at[idx_slice(b + 2)], g_refs[m], sem_g[m]
                    )

            return carry

        lax.fori_loop(0, n_blk // 2, body, 0)

        for m in range(2):
            for a in range(n_a):
                pltpu.make_async_copy(
                    t_refs[m].at[a],
                    out_hbm.at[hist - 1, a, wid * jw],
                    sem_o[m],
                ).wait()

    return gather_kernel


def kernel(X, weight):
    batch, hist = X.shape
    xt = X.T.astype(jnp.int32)
    out5 = _build_gather(hist, batch, weight.shape[0])(xt, weight)
    out = out5.transpose(2, 4, 0, 1, 3)
    return out.reshape(batch, hist, _EMBED_DIM)


# speedup vs baseline: 1.5377x; 1.5377x over previous
import functools

import jax
import jax.numpy as jnp
from jax import lax
from jax.experimental import pallas as pl
from jax.experimental.pallas import tpu as pltpu
from jax.experimental.pallas import tpu_sc as plsc

_EMBED_DIM = 64
_NUM_CORES = 2
_NUM_SUBCORES = 16
_NUM_WORKERS = _NUM_CORES * _NUM_SUBCORES
_CHUNK = 128
_NBUF = 8
_LANE = 128
_SUB = 8


@functools.lru_cache(maxsize=None)
def _build_gather(n_chunk: int, n_class: int):
    mesh = plsc.VectorSubcoreMesh(core_axis_name="c", subcore_axis_name="s")
    assert n_chunk % _NBUF == 0

    @functools.partial(
        pl.kernel,
        out_type=jax.ShapeDtypeStruct(
            (_NUM_WORKERS, n_chunk, _CHUNK, _EMBED_DIM), jnp.float32
        ),
        mesh=mesh,
        compiler_params=pltpu.CompilerParams(use_tc_tiling_on_sc=False),
        scratch_types=[
            pltpu.VMEM((n_chunk, _CHUNK), jnp.int32),
            pltpu.VMEM((_NBUF, _CHUNK, _EMBED_DIM), jnp.float32),
            pltpu.SemaphoreType.DMA((_NBUF,)),
            pltpu.SemaphoreType.DMA((_NBUF,)),
        ],
    )
    def gather_kernel(idx_hbm, table_hbm, out_hbm, idx_v, rows_v, sem_in, sem_out):
        wid = lax.axis_index("s") * _NUM_CORES + lax.axis_index("c")
        pltpu.sync_copy(idx_hbm.at[wid], idx_v)
        for b in range(_NBUF):
            pltpu.async_copy(table_hbm.at[idx_v.at[b]], rows_v.at[b], sem_in.at[b])

        def body(g, carry):
            c_base = g * _NBUF
            for b in range(_NBUF):
                c = c_base + b
                pltpu.make_async_copy(
                    table_hbm.at[idx_v.at[c]], rows_v.at[b], sem_in.at[b]
                ).wait()
                pltpu.async_copy(rows_v.at[b], out_hbm.at[wid, c], sem_out.at[b])
            for b in range(_NBUF):
                c_next = c_base + _NBUF + b

                @pl.when(c_next < n_chunk)
                def _():
                    pltpu.make_async_copy(
                        rows_v.at[b], out_hbm.at[wid, c_base + b], sem_out.at[b]
                    ).wait()
                    pltpu.async_copy(
                        table_hbm.at[idx_v.at[c_next]], rows_v.at[b], sem_in.at[b]
                    )

            return carry

        lax.fori_loop(0, n_chunk // _NBUF, body, 0)
        for b in range(_NBUF):
            pltpu.make_async_copy(
                rows_v.at[b], out_hbm.at[wid, n_chunk - _NBUF + b], sem_out.at[b]
            ).wait()

    return gather_kernel


@functools.lru_cache(maxsize=None)
def _build_format(hist: int, batch: int):
    n_jb = batch // _LANE
    grp = 16
    assert n_jb % grp == 0
    row = _LANE * _EMBED_DIM

    n_b = n_jb // grp

    def fmt_kernel(in_ref, out_ref):
        x = in_ref[0]
        x = x.reshape(grp, _LANE, _EMBED_DIM)
        x = jnp.transpose(x, (0, 2, 1))
        x = x.reshape(grp, _SUB, _SUB, _LANE)
        out_ref[0] = jnp.transpose(x, (1, 0, 2, 3))

    return pl.pallas_call(
        fmt_kernel,
        grid=(hist, n_b),
        in_specs=[pl.BlockSpec((1, grp, row), lambda h, b: (h, b, 0))],
        out_specs=pl.BlockSpec(
            (1, _SUB, grp, _SUB, _LANE), lambda h, b: (h, 0, b, 0, 0)
        ),
        out_shape=jax.ShapeDtypeStruct(
            (hist, _SUB, n_jb, _SUB, _LANE), jnp.float32
        ),
    )


def kernel(X, weight):
    batch, hist = X.shape
    n_total = batch * hist
    idx = X.T.astype(jnp.int32).reshape(-1)
    block = _NUM_WORKERS * _CHUNK
    pad = (-n_total) % block
    if pad:
        idx = jnp.concatenate([idx, jnp.zeros((pad,), jnp.int32)])
    n_chunk = (n_total + pad) // block
    idx3 = idx.reshape(_NUM_WORKERS, n_chunk, _CHUNK)
    flat = _build_gather(n_chunk, weight.shape[0])(idx3, weight)
    flat = flat.reshape(-1, _EMBED_DIM)
    if pad:
        flat = flat[:n_total]
    f3 = flat.reshape(hist, batch // _LANE, _LANE * _EMBED_DIM)
    out5 = _build_format(hist, batch)(f3)
    out = out5.transpose(2, 4, 0, 1, 3)
    return out.reshape(batch, hist, _EMBED_DIM)


# speedup vs baseline: 1.8261x; 1.1876x over previous
import functools

import jax
import jax.numpy as jnp
from jax import lax
from jax.experimental import pallas as pl
from jax.experimental.pallas import tpu as pltpu
from jax.experimental.pallas import tpu_sc as plsc

_EMBED_DIM = 64
_NUM_CORES = 2
_NUM_SUBCORES = 16
_NUM_WORKERS = _NUM_CORES * _NUM_SUBCORES
_CHUNK = 128
_NBUF = 8
_LANE = 128
_SUB = 8


@functools.lru_cache(maxsize=None)
def _build_gather(n_chunk: int, n_class: int):
    mesh = plsc.VectorSubcoreMesh(core_axis_name="c", subcore_axis_name="s")
    assert n_chunk % _NBUF == 0

    @functools.partial(
        pl.kernel,
        out_type=jax.ShapeDtypeStruct(
            (_NUM_WORKERS, n_chunk, _CHUNK, _EMBED_DIM), jnp.float32
        ),
        mesh=mesh,
        compiler_params=pltpu.CompilerParams(use_tc_tiling_on_sc=False),
        scratch_types=[
            pltpu.VMEM((n_chunk, _CHUNK), jnp.int32),
            pltpu.VMEM((_NBUF, _CHUNK, _EMBED_DIM), jnp.float32),
            pltpu.SemaphoreType.DMA((_NBUF,)),
            pltpu.SemaphoreType.DMA((_NBUF,)),
        ],
    )
    def gather_kernel(idx_hbm, table_hbm, out_hbm, idx_v, rows_v, sem_in, sem_out):
        wid = lax.axis_index("s") * _NUM_CORES + lax.axis_index("c")
        pltpu.sync_copy(idx_hbm.at[wid], idx_v)
        for b in range(_NBUF):
            pltpu.async_copy(table_hbm.at[idx_v.at[b]], rows_v.at[b], sem_in.at[b])

        def body(g, carry):
            c_base = g * _NBUF
            for b in range(_NBUF):
                c = c_base + b
                pltpu.make_async_copy(
                    table_hbm.at[idx_v.at[c]], rows_v.at[b], sem_in.at[b]
                ).wait()
                pltpu.async_copy(rows_v.at[b], out_hbm.at[wid, c], sem_out.at[b])
            for b in range(_NBUF):
                c_next = c_base + _NBUF + b

                @pl.when(c_next < n_chunk)
                def _():
                    pltpu.make_async_copy(
                        rows_v.at[b], out_hbm.at[wid, c_base + b], sem_out.at[b]
                    ).wait()
                    pltpu.async_copy(
                        table_hbm.at[idx_v.at[c_next]], rows_v.at[b], sem_in.at[b]
                    )

            return carry

        lax.fori_loop(0, n_chunk // _NBUF, body, 0)
        for b in range(_NBUF):
            pltpu.make_async_copy(
                rows_v.at[b], out_hbm.at[wid, n_chunk - _NBUF + b], sem_out.at[b]
            ).wait()

    return gather_kernel


@functools.lru_cache(maxsize=None)
def _build_format(hist: int, batch: int):
    n_jb = batch // _LANE
    grp = 16
    assert n_jb % grp == 0
    row = _LANE * _EMBED_DIM

    n_b = n_jb // grp
    half = _LANE // 2

    def fmt_kernel(in_ref, out_ref):
        x = in_ref[...]
        x = x.reshape(grp, half, _LANE)
        t = jnp.transpose(x, (0, 2, 1))
        e = t[:, :_EMBED_DIM, :]
        o = t[:, _EMBED_DIM:, :]
        y = jnp.concatenate([e, o], axis=-1)
        y = y.reshape(grp, _SUB, _SUB, _LANE)
        out_ref[0] = jnp.transpose(y, (1, 0, 2, 3))

    return pl.pallas_call(
        fmt_kernel,
        grid=(hist, n_b),
        in_specs=[
            pl.BlockSpec((grp * half, _LANE), lambda h, b: (h * n_b + b, 0))
        ],
        out_specs=pl.BlockSpec(
            (1, _SUB, grp, _SUB, _LANE), lambda h, b: (h, 0, b, 0, 0)
        ),
        out_shape=jax.ShapeDtypeStruct(
            (hist, _SUB, n_jb, _SUB, _LANE), jnp.float32
        ),
    )


def kernel(X, weight):
    batch, hist = X.shape
    n_total = batch * hist
    idx = X.T.astype(jnp.int32).reshape(-1)
    idx = idx.reshape(-1, 2, _LANE // 2).swapaxes(1, 2).reshape(-1)
    block = _NUM_WORKERS * _CHUNK
    pad = (-n_total) % block
    if pad:
        idx = jnp.concatenate([idx, jnp.zeros((pad,), jnp.int32)])
    n_chunk = (n_total + pad) // block
    idx3 = idx.reshape(_NUM_WORKERS, n_chunk, _CHUNK)
    flat = _build_gather(n_chunk, weight.shape[0])(idx3, weight)
    f2 = flat.reshape(-1, _LANE)
    if pad:
        f2 = f2[: n_total * _EMBED_DIM // _LANE]
    out5 = _build_format(hist, batch)(f2)
    out = out5.transpose(2, 4, 0, 1, 3)
    return out.reshape(batch, hist, _EMBED_DIM)
